# baseline (speedup 1.0000x reference)
import jax
import jax.numpy as jnp
from jax import lax
from jax.experimental import pallas as pl
from jax.experimental.pallas import tpu as pltpu

_B, _L, _C, _A = 32, 50, 80, 3
_IMG = 480.0
_CH = 5 + _C
_CHUNK = 1024
_LEVELS = (
    (60, 10800),
    (30, 2700),
    (15, 675),
)


def _make_level_kernel(W, N):
    Wf = float(W)
    f32 = jnp.float32
    nsub = 8

    def kern(anchors_ref, x_ref, gt_ref, loss_ref, boxes_ref, scores_ref):
        c = pl.program_id(1)

        @pl.when(c == 0)
        def _init():
            loss_ref[:, :, :] = jnp.zeros_like(loss_ref)

        xts = [jnp.transpose(x_ref[0, cs * 128:(cs + 1) * 128, :])
               for cs in range(nsub)]

        def stack(k):
            return jnp.concatenate([xt[k:k + 1, :] for xt in xts], axis=0)

        tx = stack(0)
        ty = stack(1)
        tw = stack(2)
        th = stack(3)
        tcf = stack(4)

        rows = (c * _CHUNK
                + lax.broadcasted_iota(jnp.int32, (8, 128), 0) * 128
                + lax.broadcasted_iota(jnp.int32, (8, 128), 1)).astype(f32)
        cell = jnp.floor((rows + 0.5) * (1.0 / 3.0))
        a = rows - 3.0 * cell
        iF = jnp.floor((cell + 0.5) / Wf)
        jF = cell - Wf * iF
        validc = rows < float(N)

        gt = gt_ref[0]
        gx = gt[:, 0:1]
        gy = gt[:, 1:2]
        gw = gt[:, 2:3]
        gh = gt[:, 3:4]
        gc = gt[:, 4:5]
        bw = gw * Wf
        bh = gh * Wf
        validg = bw > 0.0
        jg = jnp.clip(jnp.floor(gx * Wf), 0.0, Wf - 1.0)
        ig = jnp.clip(jnp.floor(gy * Wf), 0.0, Wf - 1.0)
        aw = [anchors_ref[k, 0] * Wf for k in range(_A)]
        ah = [anchors_ref[k, 1] * Wf for k in range(_A)]

        def anc_iou(k):
            inter = jnp.minimum(bw, aw[k]) * jnp.minimum(bh, ah[k])
            return inter / (bw * bh + aw[k] * ah[k] - inter + 1e-9)

        kb = jnp.zeros_like(gx)
        bestk = anc_iou(0)
        for k in (1, 2):
            iouk = anc_iou(k)
            upd = iouk > bestk
            kb = jnp.where(upd, float(k), kb)
            bestk = jnp.where(upd, iouk, bestk)
        anc_w = jnp.where(kb == 0.0, aw[0], jnp.where(kb == 1.0, aw[1], aw[2]))
        anc_h = jnp.where(kb == 0.0, ah[0], jnp.where(kb == 1.0, ah[1], ah[2]))
        bw_s = jnp.where(validg, bw, 1.0)
        bh_s = jnp.where(validg, bh, 1.0)

        bc = lambda v: jnp.broadcast_to(v, (_L, 128))
        jg_m = bc(jnp.where(validg, jg, -1.0))
        ig_b = bc(ig)
        kb_b = bc(kb)
        adjx = bc(gx * Wf - jg)
        adjy = bc(gy * Wf - ig)
        adjw = bc(jnp.log(bw_s / anc_w))
        adjh = bc(jnp.log(bh_s / anc_h))
        gc_b = bc(gc)
        tminx = bc(gx - gw * 0.5)
        tmaxx = bc(gx + gw * 0.5)
        tminy = bc(gy - gh * 0.5)
        tmaxy = bc(gy + gh * 0.5)
        tarea = bc(gw * gh)

        sx = jax.nn.sigmoid(tx)
        sy = jax.nn.sigmoid(ty)
        pconf = jax.nn.sigmoid(tcf)
        aw_c = jnp.where(a == 0.0, aw[0], jnp.where(a == 1.0, aw[1], aw[2]))
        ah_c = jnp.where(a == 0.0, ah[0], jnp.where(a == 1.0, ah[1], ah[2]))
        px = (sx + jF) / Wf
        py = (sy + iF) / Wf
        pw = jnp.exp(tw) * aw_c / Wf
        ph = jnp.exp(th) * ah_c / Wf
        pminx = px - pw * 0.5
        pmaxx = px + pw * 0.5
        pminy = py - ph * 0.5
        pmaxy = py + ph * 0.5
        parea = pw * ph

        best = jnp.zeros((8, 128), f32)
        maskf = jnp.zeros((8, 128), f32)
        mtbx = jnp.zeros((8, 128), f32)
        mtby = jnp.zeros((8, 128), f32)
        mtbw = jnp.zeros((8, 128), f32)
        mtbh = jnp.zeros((8, 128), f32)
        mtbc = jnp.zeros((8, 128), f32)
        for l in range(_L):
            r = lambda q: q[l:l + 1, :]
            iw = jnp.clip(jnp.minimum(pmaxx, r(tmaxx))
                          - jnp.maximum(pminx, r(tminx)), 0.0)
            ih = jnp.clip(jnp.minimum(pmaxy, r(tmaxy))
                          - jnp.maximum(pminy, r(tminy)), 0.0)
            inter = iw * ih
            iou = inter / (parea + r(tarea) - inter + 1e-9)
            best = jnp.maximum(best, iou)
            m = (jF == r(jg_m)) & (iF == r(ig_b)) & (a == r(kb_b))
            maskf = jnp.where(m, 1.0, maskf)
            mtbx = jnp.where(m, r(adjx), mtbx)
            mtby = jnp.where(m, r(adjy), mtby)
            mtbw = jnp.where(m, r(adjw), mtbw)
            mtbh = jnp.where(m, r(adjh), mtbh)
            mtbc = jnp.where(m, r(gc_b), mtbc)
        obj_det = (best > 0.6).astype(f32)

        ch_iota = lax.broadcasted_iota(jnp.int32, (_C, 1), 0).astype(f32)
        cls_rows = []
        for cs in range(nsub):
            tl = xts[cs][5:_CH, :]
            mxc = jnp.max(tl, axis=0, keepdims=True)
            e = jnp.exp(tl - mxc)
            se = jnp.sum(e, axis=0, keepdims=True)
            p = e / se
            sc = p * pconf[cs:cs + 1, :]
            scores_ref[0, cs * 128:(cs + 1) * 128, :] = jnp.transpose(sc)
            oh = (ch_iota == mtbc[cs:cs + 1, :]).astype(f32)
            d = oh - p
            cls_rows.append(jnp.sum(d * d, axis=0, keepdims=True)
                            * maskf[cs:cs + 1, :])
            bx = jnp.concatenate(
                [pminx[cs:cs + 1, :] * _IMG, pminy[cs:cs + 1, :] * _IMG,
                 pmaxx[cs:cs + 1, :] * _IMG, pmaxy[cs:cs + 1, :] * _IMG], axis=0)
            boxes_ref[0, cs * 128:(cs + 1) * 128, :] = jnp.transpose(bx)
        cls8 = jnp.concatenate(cls_rows, axis=0)

        no_obj = (1.0 - obj_det) * (1.0 - maskf) * (pconf * pconf)
        obj = 5.0 * maskf * (1.0 - pconf) ** 2
        coord = maskf * ((mtbx - sx) ** 2 + (mtby - sy) ** 2
                         + (mtbw - tw) ** 2 + (mtbh - th) ** 2)
        cells = jnp.where(validc, no_obj + obj + coord + cls8, 0.0)
        loss_ref[:, :, :] = loss_ref[:, :, :] + 0.5 * jnp.sum(cells)

    return kern


def _run_level(preds, gt_labels, anchors, W, N):
    f32 = jnp.float32
    nch = -(-N // _CHUNK)
    pin = preds.reshape(_B, N, _CH)

    loss_p, boxes, scores = pl.pallas_call(
        _make_level_kernel(W, N),
        grid=(_B, nch),
        in_specs=[
            pl.BlockSpec(memory_space=pltpu.SMEM),
            pl.BlockSpec((1, _CHUNK, _CH), lambda b, c: (b, c, 0)),
            pl.BlockSpec((1, _L, 5), lambda b, c: (b, 0, 0)),
        ],
        out_specs=[
            pl.BlockSpec((1, 1, 128), lambda b, c: (b, 0, 0)),
            pl.BlockSpec((1, _CHUNK, 4), lambda b, c: (b, c, 0)),
            pl.BlockSpec((1, _CHUNK, _C), lambda b, c: (b, c, 0)),
        ],
        out_shape=[
            jax.ShapeDtypeStruct((_B, 1, 128), f32),
            jax.ShapeDtypeStruct((_B, N, 4), f32),
            jax.ShapeDtypeStruct((_B, N, _C), f32),
        ],
        compiler_params=pltpu.CompilerParams(
            dimension_semantics=("parallel", "arbitrary")),
    )(anchors, pin, gt_labels)

    return loss_p[:, 0, 0], boxes, scores


def kernel(preds0, preds1, preds2, gt_labels, anchors):
    losses, boxes_l, scores_l = [], [], []
    for preds, (W, N) in zip((preds0, preds1, preds2), _LEVELS):
        lp, bx, sc = _run_level(preds, gt_labels, anchors, W, N)
        losses.append(lp)
        boxes_l.append(bx)
        scores_l.append(sc)
    loss = sum(jnp.sum(lp) for lp in losses)
    boxes = jnp.concatenate(boxes_l, axis=1)
    scores = jnp.concatenate(scores_l, axis=1)
    return loss, boxes, scores
